# baseline (speedup 1.0000x reference)
import jax
import jax.numpy as jnp
from jax import lax
from jax.experimental import pallas as pl
from jax.experimental.pallas import tpu as pltpu
from jax.experimental.pallas import tpu_sc as plsc

MAX_TILES = 4
LANES = 16


def _pos_body(ar_hbm, emb_hbm, pos_hbm, ar_v, idx_v, row_v):
    wid = lax.axis_index("s") * 2 + lax.axis_index("c")
    b = wid // MAX_TILES
    t = wid % MAX_TILES
    ar_v[pl.ds(LANES, LANES)] = jnp.full((LANES,), 1, jnp.int32)
    pltpu.sync_copy(ar_hbm, ar_v.at[pl.ds(0, LANES)])
    vh = ar_v[pl.ds(2 * b, LANES)]
    vw = ar_v[pl.ds(2 * b + 1, LANES)]
    rr = (jnp.where(t >= vw, 1, 0) + jnp.where(t >= 2 * vw, 1, 0)
          + jnp.where(t >= 3 * vw, 1, 0))
    cc = t - rr * vw
    fi = rr * MAX_TILES + cc
    fi = jnp.where(t < vh * vw, fi, MAX_TILES * MAX_TILES)
    idx_v[...] = fi
    pltpu.sync_copy(emb_hbm.at[idx_v.at[pl.ds(0, 1)]], row_v)
    pltpu.sync_copy(row_v, pos_hbm.at[pl.ds(wid, 1)])


def _masked_pos(ar_flat, emb_flat):
    d = emb_flat.shape[1]
    mesh = plsc.VectorSubcoreMesh(core_axis_name="c", subcore_axis_name="s")
    return pl.kernel(
        _pos_body,
        out_type=jax.ShapeDtypeStruct((2 * LANES, d), jnp.float32),
        mesh=mesh,
        scratch_types=[
            pltpu.VMEM((2 * LANES,), jnp.int32),
            pltpu.VMEM((LANES,), jnp.int32),
            pltpu.VMEM((1, d), jnp.float32),
        ],
    )(ar_flat, emb_flat)


def _add_body(gate_ref, x_ref, pos_ref, o_ref):
    g = jnp.tanh(gate_ref[0])
    o_ref[...] = x_ref[...] + g * pos_ref[...]


TOK_BLK = 208


def _broadcast_add(gate, x, pos4):
    bsz, n_tiles, n, d = x.shape
    nb = pl.cdiv(n, TOK_BLK)
    return pl.pallas_call(
        _add_body,
        grid=(bsz, n_tiles, nb),
        in_specs=[
            pl.BlockSpec(memory_space=pltpu.SMEM),
            pl.BlockSpec((1, 1, TOK_BLK, d), lambda i, j, k: (i, j, k, 0)),
            pl.BlockSpec((1, 1, 1, d), lambda i, j, k: (i, j, 0, 0)),
        ],
        out_specs=pl.BlockSpec((1, 1, TOK_BLK, d), lambda i, j, k: (i, j, k, 0)),
        out_shape=jax.ShapeDtypeStruct((bsz, n_tiles, n, d), jnp.float32),
    )(gate, x, pos4)


def kernel(x, aspect_ratio, embedding, gate):
    bsz, n_tiles, n_tok, d = x.shape
    ar_flat = aspect_ratio.astype(jnp.int32).reshape(-1)
    emb_flat = jnp.concatenate(
        [embedding.reshape(MAX_TILES * MAX_TILES, d),
         jnp.zeros((1, d), jnp.float32)], axis=0)
    pos = _masked_pos(ar_flat, emb_flat)
    return _broadcast_add(gate, x, pos.reshape(bsz, n_tiles, 1, d))


# speedup vs baseline: 1.1627x; 1.1627x over previous
import jax
import jax.numpy as jnp
from jax import lax
from jax.experimental import pallas as pl
from jax.experimental.pallas import tpu as pltpu
from jax.experimental.pallas import tpu_sc as plsc

MAX_TILES = 4
LANES = 16


def _pos_body(ar_hbm, emb_hbm, pos_hbm, ar_v, idx_v, row_v):
    wid = lax.axis_index("s") * 2 + lax.axis_index("c")
    b = wid // MAX_TILES
    t = wid % MAX_TILES
    ar_v[pl.ds(LANES, LANES)] = jnp.full((LANES,), 1, jnp.int32)
    pltpu.sync_copy(ar_hbm, ar_v.at[pl.ds(0, LANES)])
    vh = ar_v[pl.ds(2 * b, LANES)]
    vw = ar_v[pl.ds(2 * b + 1, LANES)]
    rr = (jnp.where(t >= vw, 1, 0) + jnp.where(t >= 2 * vw, 1, 0)
          + jnp.where(t >= 3 * vw, 1, 0))
    cc = t - rr * vw
    fi = rr * MAX_TILES + cc
    fi = jnp.where(t < vh * vw, fi, MAX_TILES * MAX_TILES)
    idx_v[...] = fi
    pltpu.sync_copy(emb_hbm.at[idx_v.at[pl.ds(0, 1)]], row_v)
    pltpu.sync_copy(row_v, pos_hbm.at[pl.ds(wid, 1)])


def _masked_pos(ar_flat, emb_flat):
    d = emb_flat.shape[1]
    mesh = plsc.VectorSubcoreMesh(core_axis_name="c", subcore_axis_name="s")
    return pl.kernel(
        _pos_body,
        out_type=jax.ShapeDtypeStruct((2 * LANES, d), jnp.float32),
        mesh=mesh,
        scratch_types=[
            pltpu.VMEM((2 * LANES,), jnp.int32),
            pltpu.VMEM((LANES,), jnp.int32),
            pltpu.VMEM((1, d), jnp.float32),
        ],
    )(ar_flat, emb_flat)


NBUF = 3


def _add_body(gate_ref, pos_ref, x_ref, o_ref, gp_ref, ibuf, obuf, isem, osem):
    nbt = pos_ref.shape[0]
    gp_ref[...] = jnp.tanh(gate_ref[0]) * pos_ref[...]

    def coords(p):
        b = p // MAX_TILES
        return b, p - b * MAX_TILES

    def load(i, s):
        b, t = coords(i)
        pltpu.make_async_copy(x_ref.at[b, t], ibuf.at[s], isem.at[s]).start()

    def run_chunk(g, s):
        i = g * NBUF + s

        @pl.when(i < nbt)
        def _():
            b, t = coords(i)
            pltpu.make_async_copy(x_ref.at[b, t], ibuf.at[s],
                                  isem.at[s]).wait()

            @pl.when(g > 0)
            def _():
                pltpu.make_async_copy(obuf.at[s], o_ref.at[b, t],
                                      osem.at[s]).wait()

            obuf[s, :, :] = ibuf[s, :, :] + gp_ref[pl.ds(i, 1), :]
            pltpu.make_async_copy(obuf.at[s], o_ref.at[b, t],
                                  osem.at[s]).start()

            @pl.when(i + NBUF < nbt)
            def _():
                load(i + NBUF, s)

    def body(g, carry):
        for s in range(NBUF):
            run_chunk(g, s)
        return carry

    for s in range(NBUF):
        load(s, s)
    lax.fori_loop(0, pl.cdiv(nbt, NBUF), body, 0)
    for s in range(NBUF):
        pltpu.make_async_copy(obuf.at[s], o_ref.at[0, 0], osem.at[s]).wait()


def _broadcast_add(gate, x, pos):
    bsz, n_tiles, n, d = x.shape
    return pl.pallas_call(
        _add_body,
        in_specs=[
            pl.BlockSpec(memory_space=pltpu.SMEM),
            pl.BlockSpec(memory_space=pltpu.MemorySpace.VMEM),
            pl.BlockSpec(memory_space=pltpu.MemorySpace.HBM),
        ],
        out_specs=pl.BlockSpec(memory_space=pltpu.MemorySpace.HBM),
        out_shape=jax.ShapeDtypeStruct((bsz, n_tiles, n, d), jnp.float32),
        scratch_shapes=[
            pltpu.VMEM((bsz * n_tiles, d), jnp.float32),
            pltpu.VMEM((NBUF, n, d), jnp.float32),
            pltpu.VMEM((NBUF, n, d), jnp.float32),
            pltpu.SemaphoreType.DMA((NBUF,)),
            pltpu.SemaphoreType.DMA((NBUF,)),
        ],
    )(gate, pos, x)


def kernel(x, aspect_ratio, embedding, gate):
    bsz, n_tiles, n_tok, d = x.shape
    ar_flat = aspect_ratio.astype(jnp.int32).reshape(-1)
    emb_flat = jnp.concatenate(
        [embedding.reshape(MAX_TILES * MAX_TILES, d),
         jnp.zeros((1, d), jnp.float32)], axis=0)
    pos = _masked_pos(ar_flat, emb_flat)
    return _broadcast_add(gate, x, pos)


# speedup vs baseline: 1.1635x; 1.0006x over previous
import jax
import jax.numpy as jnp
from jax import lax
from jax.experimental import pallas as pl
from jax.experimental.pallas import tpu as pltpu
from jax.experimental.pallas import tpu_sc as plsc

MAX_TILES = 4
LANES = 16


def _pos_body(ar_hbm, emb_hbm, pos_hbm, ar_v, idx_v, row_v):
    wid = lax.axis_index("s") * 2 + lax.axis_index("c")
    b = wid // MAX_TILES
    t = wid % MAX_TILES
    ar_v[pl.ds(LANES, LANES)] = jnp.full((LANES,), 1, jnp.int32)
    pltpu.sync_copy(ar_hbm, ar_v.at[pl.ds(0, LANES)])
    vh = ar_v[pl.ds(2 * b, LANES)]
    vw = ar_v[pl.ds(2 * b + 1, LANES)]
    rr = (jnp.where(t >= vw, 1, 0) + jnp.where(t >= 2 * vw, 1, 0)
          + jnp.where(t >= 3 * vw, 1, 0))
    cc = t - rr * vw
    fi = rr * MAX_TILES + cc
    fi = jnp.where(t < vh * vw, fi, MAX_TILES * MAX_TILES)
    idx_v[...] = fi
    pltpu.sync_copy(emb_hbm.at[idx_v.at[pl.ds(0, 1)]], row_v)
    pltpu.sync_copy(row_v, pos_hbm.at[pl.ds(wid, 1)])


def _masked_pos(ar_flat, emb_flat):
    d = emb_flat.shape[1]
    mesh = plsc.VectorSubcoreMesh(core_axis_name="c", subcore_axis_name="s")
    return pl.kernel(
        _pos_body,
        out_type=jax.ShapeDtypeStruct((2 * LANES, d), jnp.float32),
        mesh=mesh,
        scratch_types=[
            pltpu.VMEM((2 * LANES,), jnp.int32),
            pltpu.VMEM((LANES,), jnp.int32),
            pltpu.VMEM((1, d), jnp.float32),
        ],
    )(ar_flat, emb_flat)


NBUF = 8
COL_BLK = 256
N_COL = 5


def _add_body(gate_ref, pos_ref, x_ref, o_ref, gp_ref, ibuf, obuf, isem, osem):
    nbt = pos_ref.shape[0]
    n_chunks = nbt * N_COL
    gp_ref[...] = jnp.tanh(gate_ref[0]) * pos_ref[...]

    def coords(i):
        p = i // N_COL
        c = (i - p * N_COL) * COL_BLK
        b = p // MAX_TILES
        return p, b, p - b * MAX_TILES, c

    def load(i, s):
        _, b, t, c = coords(i)
        pltpu.make_async_copy(x_ref.at[b, t, :, pl.ds(c, COL_BLK)],
                              ibuf.at[s], isem.at[s]).start()

    def run_chunk(g, s):
        i = g * NBUF + s
        p, b, t, c = coords(i)
        pltpu.make_async_copy(x_ref.at[b, t, :, pl.ds(c, COL_BLK)],
                              ibuf.at[s], isem.at[s]).wait()

        @pl.when(g > 0)
        def _():
            pltpu.make_async_copy(obuf.at[s],
                                  o_ref.at[b, t, :, pl.ds(c, COL_BLK)],
                                  osem.at[s]).wait()

        obuf[s, :, :] = ibuf[s, :, :] + gp_ref[pl.ds(p, 1), pl.ds(c, COL_BLK)]
        pltpu.make_async_copy(obuf.at[s], o_ref.at[b, t, :, pl.ds(c, COL_BLK)],
                              osem.at[s]).start()

        @pl.when(i + NBUF < n_chunks)
        def _():
            load(i + NBUF, s)

    def body(g, carry):
        for s in range(NBUF):
            run_chunk(g, s)
        return carry

    for s in range(NBUF):
        load(s, s)
    lax.fori_loop(0, n_chunks // NBUF, body, 0)
    for s in range(NBUF):
        pltpu.make_async_copy(obuf.at[s], o_ref.at[0, 0, :, pl.ds(0, COL_BLK)],
                              osem.at[s]).wait()


def _broadcast_add(gate, x, pos):
    bsz, n_tiles, n, d = x.shape
    return pl.pallas_call(
        _add_body,
        in_specs=[
            pl.BlockSpec(memory_space=pltpu.SMEM),
            pl.BlockSpec(memory_space=pltpu.MemorySpace.VMEM),
            pl.BlockSpec(memory_space=pltpu.MemorySpace.HBM),
        ],
        out_specs=pl.BlockSpec(memory_space=pltpu.MemorySpace.HBM),
        out_shape=jax.ShapeDtypeStruct((bsz, n_tiles, n, d), jnp.float32),
        scratch_shapes=[
            pltpu.VMEM((bsz * n_tiles, d), jnp.float32),
            pltpu.VMEM((NBUF, n, COL_BLK), jnp.float32),
            pltpu.VMEM((NBUF, n, COL_BLK), jnp.float32),
            pltpu.SemaphoreType.DMA((NBUF,)),
            pltpu.SemaphoreType.DMA((NBUF,)),
        ],
    )(gate, pos, x)


def kernel(x, aspect_ratio, embedding, gate):
    bsz, n_tiles, n_tok, d = x.shape
    ar_flat = aspect_ratio.astype(jnp.int32).reshape(-1)
    emb_flat = jnp.concatenate(
        [embedding.reshape(MAX_TILES * MAX_TILES, d),
         jnp.zeros((1, d), jnp.float32)], axis=0)
    pos = _masked_pos(ar_flat, emb_flat)
    return _broadcast_add(gate, x, pos)


# speedup vs baseline: 4.0115x; 3.4479x over previous
import jax
import jax.numpy as jnp
from jax import lax
from jax.experimental import pallas as pl
from jax.experimental.pallas import tpu as pltpu
from jax.experimental.pallas import tpu_sc as plsc

MAX_TILES = 4
LANES = 16


def _pos_body(ar_hbm, emb_hbm, pos_hbm, ar_v, idx_v, row_v):
    wid = lax.axis_index("s") * 2 + lax.axis_index("c")
    b = wid // MAX_TILES
    t = wid % MAX_TILES
    ar_v[pl.ds(LANES, LANES)] = jnp.full((LANES,), 1, jnp.int32)
    pltpu.sync_copy(ar_hbm, ar_v.at[pl.ds(0, LANES)])
    vh = ar_v[pl.ds(2 * b, LANES)]
    vw = ar_v[pl.ds(2 * b + 1, LANES)]
    rr = (jnp.where(t >= vw, 1, 0) + jnp.where(t >= 2 * vw, 1, 0)
          + jnp.where(t >= 3 * vw, 1, 0))
    cc = t - rr * vw
    fi = rr * MAX_TILES + cc
    fi = jnp.where(t < vh * vw, fi, MAX_TILES * MAX_TILES)
    idx_v[...] = fi
    pltpu.sync_copy(emb_hbm.at[idx_v.at[pl.ds(0, 1)]], row_v)
    pltpu.sync_copy(row_v, pos_hbm.at[pl.ds(wid, 1)])


def _masked_pos(ar_flat, emb_flat):
    d = emb_flat.shape[1]
    mesh = plsc.VectorSubcoreMesh(core_axis_name="c", subcore_axis_name="s")
    return pl.kernel(
        _pos_body,
        out_type=jax.ShapeDtypeStruct((2 * LANES, d), jnp.float32),
        mesh=mesh,
        scratch_types=[
            pltpu.VMEM((2 * LANES,), jnp.int32),
            pltpu.VMEM((LANES,), jnp.int32),
            pltpu.VMEM((1, d), jnp.float32),
        ],
    )(ar_flat, emb_flat)


TOK_BLK = 256


def _add_body(gate_ref, x_ref, pos_ref, o_ref):
    g = jnp.tanh(gate_ref[0])
    o_ref[...] = x_ref[...] + g * pos_ref[...]


def _broadcast_add(gate, xt, pos_r):
    bsz, n, n_tiles, d = xt.shape
    nb = pl.cdiv(n, TOK_BLK)
    return pl.pallas_call(
        _add_body,
        grid=(bsz, nb),
        in_specs=[
            pl.BlockSpec(memory_space=pltpu.SMEM),
            pl.BlockSpec((1, TOK_BLK, n_tiles, d), lambda i, k: (i, k, 0, 0)),
            pl.BlockSpec((1, 1, n_tiles, d), lambda i, k: (i, 0, 0, 0)),
        ],
        out_specs=pl.BlockSpec((1, TOK_BLK, n_tiles, d),
                               lambda i, k: (i, k, 0, 0)),
        out_shape=jax.ShapeDtypeStruct((bsz, n, n_tiles, d), jnp.float32),
    )(gate, xt, pos_r)


def kernel(x, aspect_ratio, embedding, gate):
    bsz, n_tiles, n_tok, d = x.shape
    ar_flat = aspect_ratio.astype(jnp.int32).reshape(-1)
    emb_flat = jnp.concatenate(
        [embedding.reshape(MAX_TILES * MAX_TILES, d),
         jnp.zeros((1, d), jnp.float32)], axis=0)
    pos = _masked_pos(ar_flat, emb_flat)
    xt = jnp.transpose(x, (0, 2, 1, 3))
    pos_r = pos.reshape(bsz, 1, n_tiles, d)
    outt = _broadcast_add(gate, xt, pos_r)
    return jnp.transpose(outt, (0, 2, 1, 3))


# speedup vs baseline: 4.0833x; 1.0179x over previous
import jax
import jax.numpy as jnp
from jax import lax
from jax.experimental import pallas as pl
from jax.experimental.pallas import tpu as pltpu
from jax.experimental.pallas import tpu_sc as plsc

MAX_TILES = 4
LANES = 16


def _pos_body(ar_hbm, emb_hbm, pos_hbm, ar_v, idx_v, row_v):
    wid = lax.axis_index("s") * 2 + lax.axis_index("c")
    b = wid // MAX_TILES
    t = wid % MAX_TILES
    ar_v[pl.ds(LANES, LANES)] = jnp.full((LANES,), 1, jnp.int32)
    pltpu.sync_copy(ar_hbm, ar_v.at[pl.ds(0, LANES)])
    vh = ar_v[pl.ds(2 * b, LANES)]
    vw = ar_v[pl.ds(2 * b + 1, LANES)]
    rr = (jnp.where(t >= vw, 1, 0) + jnp.where(t >= 2 * vw, 1, 0)
          + jnp.where(t >= 3 * vw, 1, 0))
    cc = t - rr * vw
    fi = rr * MAX_TILES + cc
    fi = jnp.where(t < vh * vw, fi, MAX_TILES * MAX_TILES)
    idx_v[...] = fi
    pltpu.sync_copy(emb_hbm.at[idx_v.at[pl.ds(0, 1)]], row_v)
    pltpu.sync_copy(row_v, pos_hbm.at[pl.ds(wid, 1)])


def _masked_pos(ar_flat, emb_flat):
    d = emb_flat.shape[1]
    mesh = plsc.VectorSubcoreMesh(core_axis_name="c", subcore_axis_name="s")
    return pl.kernel(
        _pos_body,
        out_type=jax.ShapeDtypeStruct((2 * LANES, d), jnp.float32),
        mesh=mesh,
        scratch_types=[
            pltpu.VMEM((2 * LANES,), jnp.int32),
            pltpu.VMEM((LANES,), jnp.int32),
            pltpu.VMEM((1, d), jnp.float32),
        ],
    )(ar_flat, emb_flat)


TOK_BLK = 512


def _add_body(gate_ref, x_ref, pos_ref, o_ref):
    g = jnp.tanh(gate_ref[0])
    o_ref[...] = x_ref[...] + g * pos_ref[...]


def _broadcast_add(gate, xt, pos_r):
    bsz, n, n_tiles, d = xt.shape
    nb = pl.cdiv(n, TOK_BLK)
    return pl.pallas_call(
        _add_body,
        grid=(bsz, nb),
        in_specs=[
            pl.BlockSpec(memory_space=pltpu.SMEM),
            pl.BlockSpec((1, TOK_BLK, n_tiles, d), lambda i, k: (i, k, 0, 0)),
            pl.BlockSpec((1, 1, n_tiles, d), lambda i, k: (i, 0, 0, 0)),
        ],
        out_specs=pl.BlockSpec((1, TOK_BLK, n_tiles, d),
                               lambda i, k: (i, k, 0, 0)),
        out_shape=jax.ShapeDtypeStruct((bsz, n, n_tiles, d), jnp.float32),
    )(gate, xt, pos_r)


def kernel(x, aspect_ratio, embedding, gate):
    bsz, n_tiles, n_tok, d = x.shape
    ar_flat = aspect_ratio.astype(jnp.int32).reshape(-1)
    emb_flat = jnp.concatenate(
        [embedding.reshape(MAX_TILES * MAX_TILES, d),
         jnp.zeros((1, d), jnp.float32)], axis=0)
    pos = _masked_pos(ar_flat, emb_flat)
    xt = jnp.transpose(x, (0, 2, 1, 3))
    pos_r = pos.reshape(bsz, 1, n_tiles, d)
    outt = _broadcast_add(gate, xt, pos_r)
    return jnp.transpose(outt, (0, 2, 1, 3))
